# Initial kernel scaffold; baseline (speedup 1.0000x reference)
import jax
import jax.numpy as jnp
from jax.experimental import pallas as pl


def kernel(x, pos, lf_W1, lf_b1, lf_W2, lf_b2, m1_W1, m1_b1, m1_W2, m1_b2, m2_W1, m2_b1, m2_W2, m2_b2, ll_W, ll_b, att_W1, att_b1, att_W2, att_b2):
    raise NotImplementedError("write your pallas kernel here")


# speedup vs baseline: 3.5120x; 3.5120x over previous
import functools

import jax
import jax.numpy as jnp
from jax import lax
from jax.experimental import pallas as pl
from jax.experimental.pallas import tpu as pltpu
from jax.experimental.pallas import tpu_sc as plsc

_B, _N, _D = 4, 1024, 128
_K = 16
_GR = 64
_COUT = 384

_ROWS = 256


def _topk_body(pos_ref, posT_ref, out_ref):
    b = pl.program_id(0)
    nblk = pl.program_id(1)
    q = pos_ref[0]
    pT = posT_ref[0]
    cols = lax.broadcasted_iota(jnp.int32, (_ROWS, _N), 1)
    rows = nblk * _ROWS + lax.broadcasted_iota(jnp.int32, (_ROWS, 1), 0)
    acc = jnp.zeros((_ROWS, _N), jnp.float32)
    for c in range(3):
        diff = q[:, c:c + 1] - pT[c:c + 1, :]
        acc = acc + diff * diff
    acc = jnp.where(cols == rows, jnp.inf, acc)
    picks = []
    for _ in range(_K):
        m = jnp.min(acc, axis=1, keepdims=True)
        amin = jnp.min(jnp.where(acc == m, cols, _N), axis=1, keepdims=True)
        picks.append(amin)
        acc = jnp.where(cols == amin, jnp.inf, acc)
    idx = jnp.concatenate(picks, axis=1)
    out_ref[0] = idx + b * _N


def _topk_indices(pos):
    posT = jnp.transpose(pos, (0, 2, 1))
    return pl.pallas_call(
        _topk_body,
        grid=(_B, _N // _ROWS),
        in_specs=[
            pl.BlockSpec((1, _ROWS, 3), lambda b, n: (b, n, 0)),
            pl.BlockSpec((1, 3, _N), lambda b, n: (b, 0, 0)),
        ],
        out_specs=pl.BlockSpec((1, _ROWS, _K), lambda b, n: (b, n, 0)),
        out_shape=jax.ShapeDtypeStruct((_B, _N, _K), jnp.int32),
    )(pos, posT)


_NROWS = _B * _K * _N
_CH = 128


def _make_sc_gather():
    info = plsc.get_sparse_core_info()
    nw = info.num_cores * info.num_subcores
    rows_w = _NROWS // nw
    nch = rows_w // _CH
    mesh = plsc.VectorSubcoreMesh(core_axis_name="c", subcore_axis_name="s")

    @functools.partial(
        pl.kernel,
        mesh=mesh,
        out_type=jax.ShapeDtypeStruct((_NROWS, _D), jnp.float32),
        scratch_types=[
            pltpu.VMEM((nch, _CH), jnp.int32),
            pltpu.VMEM((_CH, _D), jnp.float32),
            pltpu.VMEM((_CH, _D), jnp.float32),
            pltpu.SemaphoreType.DMA,
            pltpu.SemaphoreType.DMA,
        ],
    )
    def gather(x_hbm, idx_hbm, out_hbm, idx_v, rows0, rows1, sem0, sem1):
        c = lax.axis_index("c")
        s = lax.axis_index("s")
        wid = s * info.num_cores + c
        base = wid * rows_w
        pltpu.sync_copy(idx_hbm.at[wid], idx_v)
        bufs = (rows0, rows1)
        sems = (sem0, sem1)
        cp0 = pltpu.async_copy(x_hbm.at[idx_v.at[0]], bufs[0], sems[0])
        cps = [cp0]
        for g in range(nch):
            if g + 1 < nch:
                cps.append(pltpu.async_copy(
                    x_hbm.at[idx_v.at[g + 1]], bufs[(g + 1) % 2],
                    sems[(g + 1) % 2]))
            cps[g].wait()
            pltpu.sync_copy(bufs[g % 2],
                            out_hbm.at[pl.ds(base + g * _CH, _CH)])

    return gather, nw, nch


def _sc_gather(x_flat, idx_flat):
    gather, nw, nch = _make_sc_gather()
    idx3d = idx_flat.reshape(nw, nch, _CH)
    return gather(x_flat, idx3d)


_BLK = 256


def _mlp_body(xg_ref, x_ref, wa_ref, wb_ref, b1_ref, w2_ref, b2_ref,
              m1w1_ref, m1b1_ref, m1w2_ref, m1b2_ref,
              m2w1_ref, m2b1_ref, m2w2_ref, m2b2_ref,
              llw_ref, llb_ref, a1_ref, a1b_ref, a2_ref, a2b_ref, out_ref):
    f32 = jnp.float32
    xb = x_ref[0]
    pre_a = jnp.dot(xb, wa_ref[...], preferred_element_type=f32) + b1_ref[...]

    def relu(v):
        return jnp.maximum(v, 0.0)

    def body(k, carry):
        mh, m1, m2, m3, nh, n1, n2, n3, den = carry
        xj = xg_ref[k]
        h1 = relu(jnp.dot(xj, wb_ref[...], preferred_element_type=f32) + pre_a)
        h = relu(jnp.dot(h1, w2_ref[...], preferred_element_type=f32) + b2_ref[...])
        y1 = jnp.concatenate([xb, h], axis=1)
        t1 = relu(jnp.dot(y1, m1w1_ref[...], preferred_element_type=f32) + m1b1_ref[...])
        z1 = relu(jnp.dot(t1, m1w2_ref[...], preferred_element_type=f32) + m1b2_ref[...])
        y2 = jnp.concatenate([y1, z1], axis=1)
        t2 = relu(jnp.dot(y2, m2w1_ref[...], preferred_element_type=f32) + m2b1_ref[...])
        z2 = relu(jnp.dot(t2, m2w2_ref[...], preferred_element_type=f32) + m2b2_ref[...])
        y3 = jnp.concatenate([y2, z2], axis=1)
        z3 = jnp.dot(y3, llw_ref[...], preferred_element_type=f32) + llb_ref[...]
        y4 = jnp.concatenate([y3, z3], axis=1)
        w = relu(jnp.dot(y4, a1_ref[...], preferred_element_type=f32) + a1b_ref[...])
        sc = jnp.sum(w * a2_ref[...], axis=1, keepdims=True) + a2b_ref[...]
        e = jnp.exp(jax.nn.sigmoid(sc))
        return (jnp.maximum(mh, h), jnp.maximum(m1, z1),
                jnp.maximum(m2, z2), jnp.maximum(m3, z3),
                nh + e * h, n1 + e * z1, n2 + e * z2, n3 + e * z3, den + e)

    neg = jnp.full((_BLK, _GR), -jnp.inf, f32)
    zer = jnp.zeros((_BLK, _GR), f32)
    init = (neg, neg, neg, neg, zer, zer, zer, zer, jnp.zeros((_BLK, 1), f32))
    mh, m1, m2, m3, nh, n1, n2, n3, den = lax.fori_loop(0, _K, body, init)
    inv = 0.5 / den
    out_ref[0] = jnp.concatenate([
        0.5 * m3 + n3 * inv, 0.5 * m2 + n2 * inv,
        0.5 * m1 + n1 * inv, 0.5 * mh + nh * inv, xb], axis=1)


def _edge_mlp(xg, x, wa, wb, b1, w2, b2, m1w1, m1b1, m1w2, m1b2,
              m2w1, m2b1, m2w2, m2b2, llw, llb, a1, a1b, a2, a2b):
    full = lambda b, n: (0, 0)
    wspec = lambda arr: pl.BlockSpec(arr.shape, full)
    return pl.pallas_call(
        _mlp_body,
        grid=(_B, _N // _BLK),
        in_specs=[
            pl.BlockSpec((_K, _BLK, _D), lambda b, n: (b, n, 0)),
            pl.BlockSpec((1, _BLK, _D), lambda b, n: (b, n, 0)),
        ] + [wspec(a) for a in (wa, wb, b1, w2, b2, m1w1, m1b1, m1w2, m1b2,
                                m2w1, m2b1, m2w2, m2b2, llw, llb, a1, a1b,
                                a2, a2b)],
        out_specs=pl.BlockSpec((1, _BLK, _COUT), lambda b, n: (b, n, 0)),
        out_shape=jax.ShapeDtypeStruct((_B, _N, _COUT), jnp.float32),
    )(xg, x, wa, wb, b1, w2, b2, m1w1, m1b1, m1w2, m1b2,
      m2w1, m2b1, m2w2, m2b2, llw, llb, a1, a1b, a2, a2b)


def kernel(x, pos, lf_W1, lf_b1, lf_W2, lf_b2, m1_W1, m1_b1, m1_W2, m1_b2,
           m2_W1, m2_b1, m2_W2, m2_b2, ll_W, ll_b,
           att_W1, att_b1, att_W2, att_b2):
    knn = _topk_indices(pos)
    idx_flat = jnp.transpose(knn, (0, 2, 1)).reshape(-1)
    xg = _sc_gather(x.reshape(_B * _N, _D), idx_flat)
    xg = xg.reshape(_B * _K, _N, _D)

    wa = lf_W1[:_D] - lf_W1[2 * _D:]
    wb = lf_W1[_D:2 * _D] + lf_W1[2 * _D:]
    g = _GR
    m1w1 = jnp.concatenate([m1_W1[g:g + _D], m1_W1[:g]], axis=0)
    m2w1 = jnp.concatenate([m2_W1[2 * g:2 * g + _D], m2_W1[g:2 * g],
                            m2_W1[:g]], axis=0)
    llw = jnp.concatenate([ll_W[3 * g:3 * g + _D], ll_W[2 * g:3 * g],
                           ll_W[g:2 * g], ll_W[:g]], axis=0)
    a1 = jnp.concatenate([att_W1[4 * g:4 * g + _D], att_W1[3 * g:4 * g],
                          att_W1[2 * g:3 * g], att_W1[g:2 * g],
                          att_W1[:g]], axis=0)
    row = lambda v: v.reshape(1, -1)
    return _edge_mlp(xg, x, wa, wb, row(lf_b1), lf_W2, row(lf_b2),
                     m1w1, row(m1_b1), m1_W2, row(m1_b2),
                     m2w1, row(m2_b1), m2_W2, row(m2_b2),
                     llw, row(ll_b), a1, row(att_b1),
                     att_W2.reshape(1, -1), att_b2.reshape(1, 1))
